# baseline (speedup 1.0000x reference)
import jax
import jax.numpy as jnp
from jax import lax
from jax.experimental import pallas as pl
from jax.experimental.pallas import tpu as pltpu
from jax.experimental.pallas import tpu_sc as plsc

N = 100000
NUM_GRAPHS = 256
EPS = 1e-05

L = 16
NC = 2
NS = 16
NW = NC * NS
UNROLL = 4
PW = 3136
PWL = N - (NW - 1) * PW
KV = PW // (L * UNROLL)
KVL = PWL // (L * UNROLL)
TVL = (PWL - KVL * L * UNROLL) // L
BINS = NUM_GRAPHS

_MESH = plsc.VectorSubcoreMesh(core_axis_name="c", subcore_axis_name="s",
                               num_cores=NC, num_subcores=NS)
_PARAMS = pltpu.CompilerParams(needs_layout_passes=False)


def _norm3(x, y, z):
    q = x * x + y * y + z * z
    i = plsc.bitcast(q, jnp.int32)
    t = plsc.bitcast(0x5F3759DF - (i >> 1), jnp.float32)
    t = t * (1.5 - 0.5 * q * t * t)
    t = t * (1.5 - 0.5 * q * t * t)
    return q * t


def _wid():
    return lax.axis_index("s") * NC + lax.axis_index("c")


def _stage_in(pos_hbm, batch_hbm, posb, bb, wid, sem):
    def issue(rows):
        cps = [pltpu.async_copy(pos_hbm.at[pl.ds(c * N + wid * PW, rows)],
                                posb.at[pl.ds(c * PW, rows)], sem)
               for c in range(3)]
        cps.append(pltpu.async_copy(batch_hbm.at[pl.ds(wid * PW, rows)],
                                    bb.at[pl.ds(0, rows)], sem))
        return cps

    @pl.when(wid < NW - 1)
    def _():
        for cp in issue(PW):
            cp.wait()

    @pl.when(wid == NW - 1)
    def _():
        for cp in issue(PWL):
            cp.wait()


def _bounds(wid):
    kv = jnp.where(wid == NW - 1, KVL, KV)
    tv = jnp.where(wid == NW - 1, TVL, 0)
    return kv, tv


def _stats_body(pos_hbm, batch_hbm, part_hbm,
                posb, bb, ps0, ps1, pc0, pc1, locb, sem):
    wid = _wid()
    iota = lax.iota(jnp.int32, L)
    zeros = jnp.zeros((L,), jnp.float32)
    ones = jnp.full((L,), 1.0, jnp.float32)
    psums = (ps0, ps1)
    pcnts = (pc0, pc1)

    def zero_body(v, _):
        for l in range(L):
            for acc in psums + pcnts:
                acc[l, pl.ds(v * L, L)] = zeros
        return 0
    lax.fori_loop(0, BINS // L, zero_body, 0)

    _stage_in(pos_hbm, batch_hbm, posb, bb, wid, sem)

    def row_vec(o, u):
        x = posb[pl.ds(o, L)]
        y = posb[pl.ds(PW + o, L)]
        z = posb[pl.ds(2 * PW + o, L)]
        nrm = _norm3(x, y, z)
        b = bb[pl.ds(o, L)]
        plsc.addupdate_scatter(psums[u % 2], [iota, b], nrm)
        plsc.addupdate_scatter(pcnts[u % 2], [iota, b], ones)

    def pass1(k, _):
        for u in range(UNROLL):
            row_vec((k * UNROLL + u) * L, u)
        return 0

    def tail1(t, _):
        row_vec((KVL * UNROLL + t) * L, 0)
        return 0

    kv, tv = _bounds(wid)
    lax.fori_loop(0, kv, pass1, 0)
    lax.fori_loop(0, tv, tail1, 0)

    def lred(v, _):
        s = ps0[0, pl.ds(v * L, L)] + ps1[0, pl.ds(v * L, L)]
        c = pc0[0, pl.ds(v * L, L)] + pc1[0, pl.ds(v * L, L)]
        for l in range(1, L):
            s = s + ps0[l, pl.ds(v * L, L)] + ps1[l, pl.ds(v * L, L)]
            c = c + pc0[l, pl.ds(v * L, L)] + pc1[l, pl.ds(v * L, L)]
        locb[pl.ds(v * L, L)] = s
        locb[pl.ds(BINS + v * L, L)] = c
        return 0
    lax.fori_loop(0, BINS // L, lred, 0)

    pltpu.sync_copy(locb, part_hbm.at[wid])


def _norm_body(pos_hbm, batch_hbm, w_hbm, part_hbm, out_hbm,
               posb, outb, bb, partb, scaleb, wv, sem):
    wid = _wid()

    cps = [pltpu.async_copy(part_hbm, partb, sem),
           pltpu.async_copy(w_hbm, wv, sem)]
    _stage_in(pos_hbm, batch_hbm, posb, bb, wid, sem)
    for cp in cps:
        cp.wait()
    wvec = wv[...]

    def gred(v, _):
        s = partb[0, pl.ds(v * L, L)]
        c = partb[0, pl.ds(BINS + v * L, L)]
        for t in range(1, NW):
            s = s + partb[t, pl.ds(v * L, L)]
            c = c + partb[t, pl.ds(BINS + v * L, L)]
        mean = s / jnp.maximum(c, 1.0)
        scaleb[pl.ds(v * L, L)] = wvec / (mean + EPS)
        return 0
    lax.fori_loop(0, BINS // L, gred, 0)

    def row_vec(o):
        b = bb[pl.ds(o, L)]
        sc = plsc.load_gather(scaleb, [b])
        for c in range(3):
            v = posb[pl.ds(c * PW + o, L)]
            outb[pl.ds(c * PW + o, L)] = v * sc

    def pass2(k, _):
        for u in range(UNROLL):
            row_vec((k * UNROLL + u) * L)
        return 0

    def tail2(t, _):
        row_vec((KVL * UNROLL + t) * L)
        return 0

    kv, tv = _bounds(wid)
    lax.fori_loop(0, kv, pass2, 0)
    lax.fori_loop(0, tv, tail2, 0)

    def stage_out(rows):
        for c in range(3):
            pltpu.sync_copy(outb.at[pl.ds(c * PW, rows)],
                            out_hbm.at[pl.ds(c * N + wid * PW, rows)])

    @pl.when(wid < NW - 1)
    def _():
        stage_out(PW)

    @pl.when(wid == NW - 1)
    def _():
        stage_out(PWL)


_stats_sc = pl.kernel(
    _stats_body,
    out_type=jax.ShapeDtypeStruct((NW, 2 * BINS), jnp.float32),
    mesh=_MESH,
    compiler_params=_PARAMS,
    scratch_types=[
        pltpu.VMEM((3 * PW,), jnp.float32),
        pltpu.VMEM((PW,), jnp.int32),
        pltpu.VMEM((L, BINS), jnp.float32),
        pltpu.VMEM((L, BINS), jnp.float32),
        pltpu.VMEM((L, BINS), jnp.float32),
        pltpu.VMEM((L, BINS), jnp.float32),
        pltpu.VMEM((2 * BINS,), jnp.float32),
        pltpu.SemaphoreType.DMA,
    ],
)

_norm_sc = pl.kernel(
    _norm_body,
    out_type=jax.ShapeDtypeStruct((3 * N,), jnp.float32),
    mesh=_MESH,
    compiler_params=_PARAMS,
    scratch_types=[
        pltpu.VMEM((3 * PW,), jnp.float32),
        pltpu.VMEM((3 * PW,), jnp.float32),
        pltpu.VMEM((PW,), jnp.int32),
        pltpu.VMEM((NW, 2 * BINS), jnp.float32),
        pltpu.VMEM((BINS,), jnp.float32),
        pltpu.VMEM((L,), jnp.float32),
        pltpu.SemaphoreType.DMA,
    ],
)


def kernel(pos, batch, weight):
    posf = pos.astype(jnp.float32).T.reshape(-1)
    batch32 = batch.astype(jnp.int32)
    wvec = jnp.full((L,), 1.0, jnp.float32) * weight[0, 0]
    part = _stats_sc(posf, batch32)
    outf = _norm_sc(posf, batch32, wvec, part)
    return outf.reshape(3, N).T


# speedup vs baseline: 1.0636x; 1.0636x over previous
import jax
import jax.numpy as jnp
from jax import lax
from jax.experimental import pallas as pl
from jax.experimental.pallas import tpu as pltpu
from jax.experimental.pallas import tpu_sc as plsc

N = 100000
NUM_GRAPHS = 256
EPS = 1e-05

L = 16
NC = 2
NS = 16
NW = NC * NS
UNROLL = 4
PW = 3136
PWL = N - (NW - 1) * PW
KV = PW // (L * UNROLL)
KVL = PWL // (L * UNROLL)
TVL = (PWL - KVL * L * UNROLL) // L
BINS = NUM_GRAPHS

_MESH = plsc.VectorSubcoreMesh(core_axis_name="c", subcore_axis_name="s",
                               num_cores=NC, num_subcores=NS)
_PARAMS = pltpu.CompilerParams(needs_layout_passes=False)


def _norm3(x, y, z):
    q = x * x + y * y + z * z
    i = plsc.bitcast(q, jnp.int32)
    t = plsc.bitcast(0x5F3759DF - (i >> 1), jnp.float32)
    t = t * (1.5 - 0.5 * q * t * t)
    t = t * (1.5 - 0.5 * q * t * t)
    return q * t


def _wid():
    return lax.axis_index("s") * NC + lax.axis_index("c")


def _stage_in(pos_hbm, batch_hbm, posb, bb, wid, sem):
    def issue(rows):
        cps = [pltpu.async_copy(pos_hbm.at[pl.ds(c * N + wid * PW, rows)],
                                posb.at[pl.ds(c * PW, rows)], sem)
               for c in range(3)]
        cps.append(pltpu.async_copy(batch_hbm.at[pl.ds(wid * PW, rows)],
                                    bb.at[pl.ds(0, rows)], sem))
        return cps

    @pl.when(wid < NW - 1)
    def _():
        for cp in issue(PW):
            cp.wait()

    @pl.when(wid == NW - 1)
    def _():
        for cp in issue(PWL):
            cp.wait()


def _bounds(wid):
    kv = jnp.where(wid == NW - 1, KVL, KV)
    tv = jnp.where(wid == NW - 1, TVL, 0)
    return kv, tv


def _stats_body(pos_hbm, batch_hbm, part_hbm,
                posb, bb, ps0, ps1, pc0, pc1, locb, sem):
    wid = _wid()
    iota = lax.iota(jnp.int32, L)
    zeros = jnp.zeros((L,), jnp.float32)
    ones = jnp.full((L,), 1.0, jnp.float32)
    psums = (ps0, ps1)
    pcnts = (pc0, pc1)

    @plsc.parallel_loop(0, BINS // L)
    def zero_body(v):
        for l in range(L):
            for acc in psums + pcnts:
                acc[l, pl.ds(v * L, L)] = zeros

    _stage_in(pos_hbm, batch_hbm, posb, bb, wid, sem)

    def row_vec(o, u):
        x = posb[pl.ds(o, L)]
        y = posb[pl.ds(PW + o, L)]
        z = posb[pl.ds(2 * PW + o, L)]
        nrm = _norm3(x, y, z)
        b = bb[pl.ds(o, L)]
        plsc.addupdate_scatter(psums[u % 2], [iota, b], nrm)
        plsc.addupdate_scatter(pcnts[u % 2], [iota, b], ones)

    def pass1(k, _):
        for u in range(UNROLL):
            row_vec((k * UNROLL + u) * L, u)
        return 0

    def tail1(t, _):
        row_vec((KVL * UNROLL + t) * L, 0)
        return 0

    kv, tv = _bounds(wid)
    lax.fori_loop(0, kv, pass1, 0)
    lax.fori_loop(0, tv, tail1, 0)

    @plsc.parallel_loop(0, BINS // L)
    def lred(v):
        s = ps0[0, pl.ds(v * L, L)] + ps1[0, pl.ds(v * L, L)]
        c = pc0[0, pl.ds(v * L, L)] + pc1[0, pl.ds(v * L, L)]
        for l in range(1, L):
            s = s + ps0[l, pl.ds(v * L, L)] + ps1[l, pl.ds(v * L, L)]
            c = c + pc0[l, pl.ds(v * L, L)] + pc1[l, pl.ds(v * L, L)]
        locb[pl.ds(v * L, L)] = s
        locb[pl.ds(BINS + v * L, L)] = c

    pltpu.sync_copy(locb, part_hbm.at[wid])


def _norm_body(pos_hbm, batch_hbm, w_hbm, part_hbm, out_hbm,
               posb, outb, bb, partb, scaleb, wv, sem):
    wid = _wid()

    cps = [pltpu.async_copy(part_hbm, partb, sem),
           pltpu.async_copy(w_hbm, wv, sem)]
    _stage_in(pos_hbm, batch_hbm, posb, bb, wid, sem)
    for cp in cps:
        cp.wait()
    wvec = wv[...]

    @plsc.parallel_loop(0, BINS // L)
    def gred(v):
        s = partb[0, pl.ds(v * L, L)]
        c = partb[0, pl.ds(BINS + v * L, L)]
        for t in range(1, NW):
            s = s + partb[t, pl.ds(v * L, L)]
            c = c + partb[t, pl.ds(BINS + v * L, L)]
        mean = s / jnp.maximum(c, 1.0)
        scaleb[pl.ds(v * L, L)] = wvec / (mean + EPS)

    def row_vec(o):
        b = bb[pl.ds(o, L)]
        sc = plsc.load_gather(scaleb, [b])
        for c in range(3):
            v = posb[pl.ds(c * PW + o, L)]
            outb[pl.ds(c * PW + o, L)] = v * sc

    kv, tv = _bounds(wid)

    @plsc.parallel_loop(0, kv * UNROLL, unroll=UNROLL)
    def pass2(k):
        row_vec(k * L)

    def tail2(t, _):
        row_vec((KVL * UNROLL + t) * L)
        return 0
    lax.fori_loop(0, tv, tail2, 0)

    def stage_out(rows):
        for c in range(3):
            pltpu.sync_copy(outb.at[pl.ds(c * PW, rows)],
                            out_hbm.at[pl.ds(c * N + wid * PW, rows)])

    @pl.when(wid < NW - 1)
    def _():
        stage_out(PW)

    @pl.when(wid == NW - 1)
    def _():
        stage_out(PWL)


_stats_sc = pl.kernel(
    _stats_body,
    out_type=jax.ShapeDtypeStruct((NW, 2 * BINS), jnp.float32),
    mesh=_MESH,
    compiler_params=_PARAMS,
    scratch_types=[
        pltpu.VMEM((3 * PW,), jnp.float32),
        pltpu.VMEM((PW,), jnp.int32),
        pltpu.VMEM((L, BINS), jnp.float32),
        pltpu.VMEM((L, BINS), jnp.float32),
        pltpu.VMEM((L, BINS), jnp.float32),
        pltpu.VMEM((L, BINS), jnp.float32),
        pltpu.VMEM((2 * BINS,), jnp.float32),
        pltpu.SemaphoreType.DMA,
    ],
)

_norm_sc = pl.kernel(
    _norm_body,
    out_type=jax.ShapeDtypeStruct((3 * N,), jnp.float32),
    mesh=_MESH,
    compiler_params=_PARAMS,
    scratch_types=[
        pltpu.VMEM((3 * PW,), jnp.float32),
        pltpu.VMEM((3 * PW,), jnp.float32),
        pltpu.VMEM((PW,), jnp.int32),
        pltpu.VMEM((NW, 2 * BINS), jnp.float32),
        pltpu.VMEM((BINS,), jnp.float32),
        pltpu.VMEM((L,), jnp.float32),
        pltpu.SemaphoreType.DMA,
    ],
)


def kernel(pos, batch, weight):
    posf = pos.astype(jnp.float32).T.reshape(-1)
    batch32 = batch.astype(jnp.int32)
    wvec = jnp.full((L,), 1.0, jnp.float32) * weight[0, 0]
    part = _stats_sc(posf, batch32)
    outf = _norm_sc(posf, batch32, wvec, part)
    return outf.reshape(3, N).T
